# baseline (speedup 1.0000x reference)
import functools

import jax
import jax.numpy as jnp
from jax import lax
from jax.experimental import pallas as pl
from jax.experimental.pallas import tpu as pltpu
from jax.experimental.pallas import tpu_sc as plsc

B = 16384
EMB = 64
NUM_CORES = 2
NUM_SUBCORES = 16
NW = NUM_CORES * NUM_SUBCORES
BPW = B // NW


def _sc_gather_gmf(uid, iid, Ug, Ig, Um, Im):
    mesh = plsc.VectorSubcoreMesh(core_axis_name="c", subcore_axis_name="s")

    @functools.partial(
        pl.kernel,
        mesh=mesh,
        compiler_params=pltpu.CompilerParams(use_tc_tiling_on_sc=False),
        out_type=[
            jax.ShapeDtypeStruct((B, EMB), jnp.float32),
            jax.ShapeDtypeStruct((B, EMB), jnp.float32),
            jax.ShapeDtypeStruct((B, EMB), jnp.float32),
        ],
        scratch_types=[
            pltpu.VMEM((BPW,), jnp.int32),
            pltpu.VMEM((BPW,), jnp.int32),
            pltpu.VMEM((BPW, EMB), jnp.float32),
            pltpu.VMEM((BPW, EMB), jnp.float32),
            pltpu.SemaphoreType.DMA,
            pltpu.SemaphoreType.DMA,
        ],
    )
    def k(uid_hbm, iid_hbm, ug_hbm, ig_hbm, um_hbm, im_hbm,
          gmf_out, um_out, im_out, idx_u, idx_i, buf_a, buf_b, sem_a, sem_b):
        wid = lax.axis_index("s") * NUM_CORES + lax.axis_index("c")
        base = wid * BPW
        pltpu.sync_copy(uid_hbm.at[pl.ds(base, BPW)], idx_u)
        pltpu.sync_copy(iid_hbm.at[pl.ds(base, BPW)], idx_i)
        cp_a = pltpu.async_copy(ug_hbm.at[idx_u], buf_a, sem_a)
        cp_b = pltpu.async_copy(ig_hbm.at[idx_i], buf_b, sem_b)
        cp_a.wait()
        cp_b.wait()

        def mul_row(i, carry):
            for j in range(EMB // 16):
                sl = pl.ds(j * 16, 16)
                buf_a[i, sl] = buf_a[i, sl] * buf_b[i, sl]
            return carry

        lax.fori_loop(0, BPW, mul_row, 0)
        pltpu.sync_copy(buf_a, gmf_out.at[pl.ds(base, BPW)])
        cp_a = pltpu.async_copy(um_hbm.at[idx_u], buf_a, sem_a)
        cp_b = pltpu.async_copy(im_hbm.at[idx_i], buf_b, sem_b)
        cp_a.wait()
        cp_b.wait()
        pltpu.sync_copy(buf_a, um_out.at[pl.ds(base, BPW)])
        pltpu.sync_copy(buf_b, im_out.at[pl.ds(base, BPW)])

    return k(uid, iid, Ug, Ig, Um, Im)


def _tc_mlp(gmf, um, im, W1, b1, W2, b2, W3, b3, Wp, bp):
    w1u = W1[:, :EMB].T
    w1i = W1[:, EMB:].T
    w2 = W2.T
    w3 = W3.T
    wpg = Wp[:, :EMB]
    wpx = Wp[:, EMB:]
    b1r = b1.reshape(1, -1)
    b2r = b2.reshape(1, -1)
    b3r = b3.reshape(1, -1)
    bpr = jnp.reshape(bp, (1, 1))

    BLK = 2048
    h0 = W1.shape[0]
    h1 = W2.shape[0]
    h2 = W3.shape[0]

    def body(gmf_ref, um_ref, im_ref, w1u_ref, w1i_ref, b1_ref, w2_ref,
             b2_ref, w3_ref, b3_ref, wpg_ref, wpx_ref, bp_ref, out_ref):
        x = jnp.dot(um_ref[...], w1u_ref[...], preferred_element_type=jnp.float32)
        x = x + jnp.dot(im_ref[...], w1i_ref[...], preferred_element_type=jnp.float32)
        x = jnp.maximum(x + b1_ref[...], 0.0)
        x = jnp.dot(x, w2_ref[...], preferred_element_type=jnp.float32)
        x = jnp.maximum(x + b2_ref[...], 0.0)
        x = jnp.dot(x, w3_ref[...], preferred_element_type=jnp.float32)
        x = jnp.maximum(x + b3_ref[...], 0.0)
        logit = (jnp.sum(gmf_ref[...] * wpg_ref[...], axis=1, keepdims=True)
                 + jnp.sum(x * wpx_ref[...], axis=1, keepdims=True)
                 + bp_ref[0, 0])
        out_ref[...] = 1.0 / (1.0 + jnp.exp(-logit))

    full = lambda r, c: pl.BlockSpec((r, c), lambda i: (0, 0))
    out = pl.pallas_call(
        body,
        grid=(B // BLK,),
        in_specs=[
            pl.BlockSpec((BLK, EMB), lambda i: (i, 0)),
            pl.BlockSpec((BLK, EMB), lambda i: (i, 0)),
            pl.BlockSpec((BLK, EMB), lambda i: (i, 0)),
            full(EMB, h0),
            full(EMB, h0),
            full(1, h0),
            full(h0, h1),
            full(1, h1),
            full(h1, h2),
            full(1, h2),
            full(1, EMB),
            full(1, h2),
            full(1, 1),
        ],
        out_specs=pl.BlockSpec((BLK, 1), lambda i: (i, 0)),
        out_shape=jax.ShapeDtypeStruct((B, 1), jnp.float32),
    )(gmf, um, im, w1u, w1i, b1r, w2, b2r, w3, b3r, wpg, wpx, bpr)
    return jnp.squeeze(out, axis=-1)


def kernel(user_ids, item_ids, Ug, Ig, Um, Im, W1, b1, W2, b2, W3, b3, Wp, bp):
    uid = user_ids.astype(jnp.int32)
    iid = item_ids.astype(jnp.int32)
    gmf, um, im = _sc_gather_gmf(uid, iid, Ug, Ig, Um, Im)
    return _tc_mlp(gmf, um, im, W1, b1, W2, b2, W3, b3, Wp, bp)
